# Initial kernel scaffold; baseline (speedup 1.0000x reference)
import jax
import jax.numpy as jnp
from jax.experimental import pallas as pl


def kernel(belief, probability, payoff_sample, edge_index):
    raise NotImplementedError("write your pallas kernel here")


# speedup vs baseline: 1367.5863x; 1367.5863x over previous
import functools

import jax
import jax.numpy as jnp
from jax import lax
from jax.experimental import pallas as pl
from jax.experimental.pallas import tpu as pltpu
from jax.experimental.pallas import tpu_sc as plsc

L = 16
NS = 16
NC = 2
NW = NC * NS
CHUNK = 2048
ROWS = CHUNK // 128


def _prep_body(b_ref, p_ref, v_ref):
    mask = b_ref[...] > 0.5
    packed = (p_ref[...] << 16) | 1
    v_ref[...] = jnp.where(mask, packed, 0)


def _apply_body(b_ref, q_ref, a0_ref, a1_ref, c0_ref, c1_ref, o_ref):
    b = b_ref[...]
    q = q_ref[...]
    a = (a0_ref[...] + a1_ref[...]).astype(jnp.float32)
    cnt = c0_ref[...] + c1_ref[...]
    t = jnp.exp(a * jnp.log((1.0 - q) / q))
    den = b + (1.0 - b) * t
    post = jnp.where(den > 0.0, b / den, b)
    o_ref[...] = jnp.where(cnt > 0, post, b)


def _make_sc_kernel(n_pad, e_pad):
    sl = n_pad // NS
    w_edges = e_pad // NW
    chunks = w_edges // CHUNK
    mesh = plsc.VectorSubcoreMesh(
        core_axis_name="c", subcore_axis_name="s",
        num_cores=NC, num_subcores=NS)

    @functools.partial(
        pl.kernel,
        out_type=(jax.ShapeDtypeStruct((NC, n_pad), jnp.int32),
                  jax.ShapeDtypeStruct((NC, n_pad), jnp.int32)),
        mesh=mesh,
        compiler_params=pltpu.CompilerParams(needs_layout_passes=False),
        scratch_types=[
            pltpu.VMEM((n_pad,), jnp.int32),
            pltpu.VMEM((CHUNK,), jnp.int32),
            pltpu.VMEM((ROWS, 128), jnp.int32),
            pltpu.VMEM((CHUNK,), jnp.int32),
            pltpu.VMEM((CHUNK,), jnp.int32),
            pltpu.VMEM((sl,), jnp.int32),
            pltpu.VMEM_SHARED((n_pad,), jnp.int32),
            pltpu.VMEM_SHARED((n_pad,), jnp.int32),
        ],
    )
    def sc_kernel(vpk_hbm, src_hbm, dst_hbm, aout, cout,
                  table, six, dix, aval, cval, zbuf, acc_a, acc_c):
        i32 = jnp.int32
        c = lax.axis_index("c")
        s = lax.axis_index("s")
        w = c * i32(NS) + s

        def zero_body(i, carry):
            zbuf[pl.ds(i * i32(L), L)] = jnp.zeros((L,), jnp.int32)
            return carry
        lax.fori_loop(jnp.int32(0), jnp.int32(sl // L), zero_body, jnp.int32(0))
        pltpu.sync_copy(zbuf, acc_a.at[pl.ds(s * i32(sl), sl)])
        pltpu.sync_copy(zbuf, acc_c.at[pl.ds(s * i32(sl), sl)])
        pltpu.sync_copy(vpk_hbm, table)
        plsc.subcore_barrier()

        for k in range(chunks):
            base = w * i32(w_edges) + i32(k * CHUNK)
            pltpu.sync_copy(src_hbm.at[pl.ds(base, CHUNK)], six)
            pltpu.sync_copy(
                dst_hbm.at[pl.ds(w * i32(w_edges // 128) + i32(k * ROWS), ROWS)], dix)

            def msg_body(i, carry):
                idx = six[pl.ds(i * i32(L), L)]
                v = plsc.load_gather(table, [idx])
                cnt = v & 0xFFFF
                aval[pl.ds(i * i32(L), L)] = 2 * (v >> 16) - 10 * cnt
                cval[pl.ds(i * i32(L), L)] = cnt
                return carry
            lax.fori_loop(jnp.int32(0), jnp.int32(CHUNK // L), msg_body, jnp.int32(0))

            for j in range(ROWS):
                pltpu.sync_copy(aval.at[pl.ds(jnp.int32(j * 128), 128)],
                                acc_a.at[dix.at[jnp.int32(j)]], add=True)
                pltpu.sync_copy(cval.at[pl.ds(jnp.int32(j * 128), 128)],
                                acc_c.at[dix.at[jnp.int32(j)]], add=True)

        plsc.subcore_barrier()
        pltpu.sync_copy(acc_a.at[pl.ds(s * i32(sl), sl)],
                        aout.at[c, pl.ds(s * i32(sl), sl)])
        pltpu.sync_copy(acc_c.at[pl.ds(s * i32(sl), sl)],
                        cout.at[c, pl.ds(s * i32(sl), sl)])

    return sc_kernel


def kernel(belief, probability, payoff_sample, edge_index):
    n = belief.shape[0]
    e = edge_index.shape[1]
    n_pad = -(-n // (NS * 128)) * (NS * 128)
    e_pad = -(-e // (NW * CHUNK)) * (NW * CHUNK)
    rows2d = n_pad // 128

    b32 = belief.astype(jnp.float32)
    q32 = probability.astype(jnp.float32)
    p32 = payoff_sample.astype(jnp.int32)
    src32 = edge_index[0].astype(jnp.int32)
    dst32 = edge_index[1].astype(jnp.int32)

    b_pad = jnp.pad(b32, (0, n_pad - n)).reshape(rows2d, 128)
    q_pad = jnp.pad(q32, (0, n_pad - n), constant_values=0.5).reshape(rows2d, 128)
    p_pad = jnp.pad(p32, (0, n_pad - n)).reshape(rows2d, 128)
    src_pad = jnp.pad(src32, (0, e_pad - e), constant_values=n)
    dst_pad = jnp.pad(dst32, (0, e_pad - e), constant_values=n).reshape(
        e_pad // 128, 128)

    vpk = pl.pallas_call(
        _prep_body,
        out_shape=jax.ShapeDtypeStruct((rows2d, 128), jnp.int32),
    )(b_pad, p_pad)

    acc_a, acc_c = _make_sc_kernel(n_pad, e_pad)(
        vpk.reshape(n_pad), src_pad, dst_pad)

    out = pl.pallas_call(
        _apply_body,
        out_shape=jax.ShapeDtypeStruct((rows2d, 128), jnp.float32),
    )(b_pad, q_pad,
      acc_a[0].reshape(rows2d, 128), acc_a[1].reshape(rows2d, 128),
      acc_c[0].reshape(rows2d, 128), acc_c[1].reshape(rows2d, 128))

    return out.reshape(n_pad)[:n].astype(jnp.float64)
